# baseline (device time: 16923 ns/iter reference)
import jax
import jax.numpy as jnp
from jax import lax
from jax.experimental import pallas as pl
from jax.experimental.pallas import tpu as pltpu

N_PARTNERS = 7
NQ = 4


def kernel(x):
    m, n = x.shape[-2], x.shape[-1]
    qr = m // NQ
    x2 = x.reshape(m, n)

    def body(
        x_ref,
        out_ref,
        acc0,
        acc1,
        acc2,
        recv_x,
        recv1,
        recv2,
        sem_x,
        send1,
        send2,
        rsem1,
        rsem2,
    ):
        my_x = lax.axis_index("x")
        my_y = lax.axis_index("y")
        my_z = lax.axis_index("z")
        me = (my_x, my_y, my_z)

        def y_at(off):
            return (my_x, jnp.bitwise_and(my_y + off, 3), my_z)

        def z_at(off):
            return (my_x, my_y, jnp.bitwise_and(my_z + off, 3))

        partners = [(1 - my_x, my_y, my_z)]
        partners += [y_at(j) for j in (1, 2, 3)]
        partners += [z_at(j) for j in (1, 2, 3)]

        barrier_sem = pltpu.get_barrier_semaphore()
        for p in partners:
            pl.semaphore_signal(
                barrier_sem, inc=1,
                device_id=p, device_id_type=pl.DeviceIdType.MESH,
            )
        acc0[...] = x_ref[...].astype(jnp.bfloat16)
        pl.semaphore_wait(barrier_sem, N_PARTNERS)

        def rows(q):
            return pl.ds(q * qr, qr)

        ax1 = {0: y_at, 1: y_at, 2: z_at, 3: z_at}
        ax2 = {0: z_at, 1: z_at, 2: y_at, 3: y_at}
        ORDER = (0, 2, 1, 3)

        def bcast(src, dst_slots, send_sems, recv_sems, at):
            rdmas = []
            for j in (1, 2, 3):
                r = pltpu.make_async_remote_copy(
                    src_ref=src,
                    dst_ref=dst_slots.at[4 - j],
                    send_sem=send_sems.at[j - 1],
                    recv_sem=recv_sems.at[4 - j],
                    device_id=at(j),
                    device_id_type=pl.DeviceIdType.MESH,
                )
                r.start()
                rdmas.append(r)
            return rdmas

        def wait_recvs(slots, recv_sems):
            for s in (1, 2, 3):
                r = pltpu.make_async_remote_copy(
                    src_ref=slots.at[s],
                    dst_ref=slots.at[s],
                    send_sem=sem_x.at[0, 0],
                    recv_sem=recv_sems.at[s],
                    device_id=me,
                    device_id_type=pl.DeviceIdType.MESH,
                )
                r.wait_recv()

        drain = []

        xch = []
        for q in ORDER:
            r = pltpu.make_async_remote_copy(
                src_ref=acc0.at[rows(q)],
                dst_ref=recv_x.at[rows(q)],
                send_sem=sem_x.at[q, 0],
                recv_sem=sem_x.at[q, 1],
                device_id=partners[0],
                device_id_type=pl.DeviceIdType.MESH,
            )
            r.start()
            xch.append((q, r))
        drain += [r for _, r in xch]

        s1 = {}
        for q, r in xch:
            r.wait_recv()
            acc1[rows(q), :] = acc0[rows(q), :] + recv_x[rows(q), :]
            s1[q] = bcast(
                acc1.at[rows(q)], recv1.at[q], send1.at[q], rsem1.at[q],
                ax1[q],
            )
            drain += s1[q]

        for q in ORDER:
            wait_recvs(recv1.at[q], rsem1.at[q])
            acc2[rows(q), :] = (
                acc1[rows(q), :] + recv1[q, 1] + recv1[q, 2] + recv1[q, 3]
            )
            s2 = bcast(
                acc2.at[rows(q)], recv2.at[q], send2.at[q], rsem2.at[q],
                ax2[q],
            )
            drain += s2

        for q in ORDER:
            wait_recvs(recv2.at[q], rsem2.at[q])
            out_ref[rows(q), :] = (
                acc2[rows(q), :] + recv2[q, 1] + recv2[q, 2] + recv2[q, 3]
            ).astype(jnp.float32)

        for r in drain:
            r.wait_send()

    return pl.pallas_call(
        body,
        out_shape=jax.ShapeDtypeStruct((m, n), jnp.float32),
        in_specs=[pl.BlockSpec(memory_space=pltpu.VMEM)],
        out_specs=pl.BlockSpec(memory_space=pltpu.VMEM),
        scratch_shapes=[
            pltpu.VMEM((m, n), jnp.bfloat16),
            pltpu.VMEM((m, n), jnp.bfloat16),
            pltpu.VMEM((m, n), jnp.bfloat16),
            pltpu.VMEM((m, n), jnp.bfloat16),
            pltpu.VMEM((NQ, 4, qr, n), jnp.bfloat16),
            pltpu.VMEM((NQ, 4, qr, n), jnp.bfloat16),
            pltpu.SemaphoreType.DMA((NQ, 2)),
            pltpu.SemaphoreType.DMA((NQ, 3)),
            pltpu.SemaphoreType.DMA((NQ, 3)),
            pltpu.SemaphoreType.DMA((NQ, 4)),
            pltpu.SemaphoreType.DMA((NQ, 4)),
        ],
        compiler_params=pltpu.CompilerParams(collective_id=0),
    )(x2)
